# baseline (speedup 1.0000x reference)
import functools

import jax
import jax.numpy as jnp
from jax import lax
from jax.experimental import pallas as pl
from jax.experimental.pallas import tpu as pltpu
from jax.experimental.pallas import tpu_sc as plsc

B, W, H, C = 8, 224, 224, 96
HC = H * C
LANES = 16
ROWS, COLS = 168, 128
RCHUNK = COLS // LANES
NWORK = 32
NTASK = B * C
TPW = NTASK // NWORK

_mesh = plsc.VectorSubcoreMesh(core_axis_name="c", subcore_axis_name="s")


@functools.partial(
    pl.kernel,
    mesh=_mesh,
    out_type=jax.ShapeDtypeStruct((NTASK * 2,), jnp.float32),
    scratch_types=[
        pltpu.VMEM((ROWS, COLS), jnp.float32),
        pltpu.VMEM((ROWS, COLS), jnp.float32),
        pltpu.VMEM((2 * TPW,), jnp.float32),
        pltpu.SemaphoreType.DMA,
        pltpu.SemaphoreType.DMA,
    ],
)
def _argmax_sc(x_hbm, out_hbm, buf0, buf1, obuf, sem0, sem1):
    cid = lax.axis_index("c")
    sid = lax.axis_index("s")
    wid = sid * 2 + cid
    t0 = wid * TPW

    bufs = (buf0, buf1)
    sems = (sem0, sem1)

    def start_copy(k):
        t = t0 + k
        return pltpu.async_copy(x_hbm.at[t // C, t % C], bufs[k % 2], sems[k % 2])

    copies = [start_copy(0), None]
    lanes = lax.iota(jnp.int32, LANES)
    acc = jnp.zeros((LANES,), jnp.float32)

    for k in range(TPW):
        if k + 1 < TPW:
            copies[(k + 1) % 2] = start_copy(k + 1)
        copies[k % 2].wait()
        buf = bufs[k % 2]

        def step(r, carry):
            new = []
            for u in range(RCHUNK):
                m, rk = carry[u]
                v = buf[r, pl.ds(u * LANES, LANES)]
                gt = v > m
                m = jnp.where(gt, v, m)
                rk = jnp.where(gt, r * RCHUNK + u, rk)
                new.append((m, rk))
            return tuple(new)

        m0 = jnp.full((LANES,), -jnp.inf, jnp.float32)
        rk0 = jnp.zeros((LANES,), jnp.int32)
        accs = lax.fori_loop(0, ROWS, step, tuple((m0, rk0) for _ in range(RCHUNK)))

        m, rk = accs[0]
        for u in range(1, RCHUNK):
            mu, rku = accs[u]
            take = jnp.logical_or(mu > m, jnp.logical_and(mu == m, rku < rk))
            m = jnp.where(take, mu, m)
            rk = jnp.where(take, rku, rk)

        idx = rk * LANES + lanes
        bv, bi = m[0], idx[0]
        for l in range(1, LANES):
            v, i = m[l], idx[l]
            take = jnp.logical_or(v > bv, jnp.logical_and(v == bv, i < bi))
            bv = jnp.where(take, v, bv)
            bi = jnp.where(take, i, bi)
        gidx = bi
        colf = (gidx % W).astype(jnp.float32)
        rowf = (gidx // W).astype(jnp.float32)
        p = (2 * k) % LANES
        acc = jnp.where(lanes == p, colf, acc)
        acc = jnp.where(lanes == p + 1, rowf, acc)
        if p + 2 == LANES:
            obuf[pl.ds((k // (LANES // 2)) * LANES, LANES)] = acc

    pltpu.sync_copy(obuf, out_hbm.at[pl.ds(t0 * 2, 2 * TPW)])


def kernel(x):
    y = x[:, :C].reshape(B, C, ROWS, COLS)
    out = _argmax_sc(y)
    return out.reshape(B, 2 * C)


# speedup vs baseline: 7.3304x; 7.3304x over previous
import functools

import jax
import jax.numpy as jnp
from jax import lax
from jax.experimental import pallas as pl
from jax.experimental.pallas import tpu as pltpu
from jax.experimental.pallas import tpu_sc as plsc

B, W, H, C = 8, 224, 224, 96
LANES = 16
HCHUNK = H // LANES
NWORK = 32
NTASK = B * C
TPW = NTASK // NWORK

_mesh = plsc.VectorSubcoreMesh(core_axis_name="c", subcore_axis_name="s")


@functools.partial(
    pl.kernel,
    mesh=_mesh,
    out_type=jax.ShapeDtypeStruct((NTASK * 2,), jnp.float32),
    scratch_types=[
        pltpu.VMEM((C, H), jnp.float32),
        pltpu.VMEM((C, H), jnp.float32),
        pltpu.VMEM((2 * TPW,), jnp.float32),
        pltpu.SemaphoreType.DMA,
        pltpu.SemaphoreType.DMA,
    ],
)
def _argmax_sc(xt_hbm, out_hbm, buf0, buf1, obuf, sem0, sem1):
    cid = lax.axis_index("c")
    sid = lax.axis_index("s")
    wid = sid * 2 + cid
    t0 = wid * TPW

    bufs = (buf0, buf1)
    sems = (sem0, sem1)

    def start_copy(k):
        t = t0 + k
        return pltpu.async_copy(xt_hbm.at[t // C, t % C], bufs[k % 2], sems[k % 2])

    copies = [start_copy(0), None]
    lanes = lax.iota(jnp.int32, LANES)
    acc = jnp.zeros((LANES,), jnp.float32)

    for k in range(TPW):
        if k + 1 < TPW:
            copies[(k + 1) % 2] = start_copy(k + 1)
        copies[k % 2].wait()
        buf = bufs[k % 2]

        def step(c, carry):
            new = []
            for g in range(HCHUNK):
                m, rc = carry[g]
                v = buf[c, pl.ds(g * LANES, LANES)]
                gt = v > m
                m = jnp.where(gt, v, m)
                rc = jnp.where(gt, c, rc)
                new.append((m, rc))
            return tuple(new)

        m0 = jnp.full((LANES,), -jnp.inf, jnp.float32)
        rc0 = jnp.zeros((LANES,), jnp.int32)
        accs = lax.fori_loop(0, C, step, tuple((m0, rc0) for _ in range(HCHUNK)))

        m, rc = accs[0]
        flat = rc + lanes * C
        for g in range(1, HCHUNK):
            mg, rcg = accs[g]
            fg = rcg + (lanes * C + g * (LANES * C))
            take = jnp.logical_or(mg > m, jnp.logical_and(mg == m, fg < flat))
            m = jnp.where(take, mg, m)
            flat = jnp.where(take, fg, flat)

        bv, bi = m[0], flat[0]
        for l in range(1, LANES):
            v, i = m[l], flat[l]
            take = jnp.logical_or(v > bv, jnp.logical_and(v == bv, i < bi))
            bv = jnp.where(take, v, bv)
            bi = jnp.where(take, i, bi)
        gidx = bi
        colf = (gidx % W).astype(jnp.float32)
        rowf = (gidx // W).astype(jnp.float32)
        p = (2 * k) % LANES
        acc = jnp.where(lanes == p, colf, acc)
        acc = jnp.where(lanes == p + 1, rowf, acc)
        if p + 2 == LANES:
            obuf[pl.ds((k // (LANES // 2)) * LANES, LANES)] = acc

    pltpu.sync_copy(obuf, out_hbm.at[pl.ds(t0 * 2, 2 * TPW)])


def kernel(x):
    xt = jnp.transpose(x, (0, 1, 3, 2))
    out = _argmax_sc(xt)
    return out.reshape(B, 2 * C)
